# baseline (speedup 1.0000x reference)
import functools

import jax
import jax.numpy as jnp
from jax.experimental import pallas as pl
from jax.experimental.pallas import tpu as pltpu

_E = 8
_IN_CH = 96
_OUT_CH = 96
_KDIM = 9 * _IN_CH
_OH = 111
_NPIX = _OH * _OH
_NPAD = 12800
_TILE_N = 1280


def _routing_body(rv_ref, w1_ref, b1_ref, w2_ref, b2_ref, emb_ref, idx_ref):
    h = jnp.dot(rv_ref[...], w1_ref[...], preferred_element_type=jnp.float32)
    h = jnp.maximum(h + b1_ref[...], 0.0)
    r = jnp.dot(h, w2_ref[...], preferred_element_type=jnp.float32) + b2_ref[...]
    rn = r / jnp.maximum(jnp.sqrt(jnp.sum(r * r, axis=-1, keepdims=True)), 1e-8)
    e = emb_ref[...]
    en = e / jnp.maximum(jnp.sqrt(jnp.sum(e * e, axis=-1, keepdims=True)), 1e-8)
    sim = jax.lax.dot_general(rn, en, (((1,), (1,)), ((), ())),
                              preferred_element_type=jnp.float32)
    m = jnp.max(sim, axis=-1, keepdims=True)
    iota = jax.lax.broadcasted_iota(jnp.int32, sim.shape, 1)
    cand = jnp.where(sim >= m, iota, jnp.int32(_E))
    idx_ref[...] = jnp.min(cand, axis=-1, keepdims=True)


def _conv_body(idx_ref, x_ref, w_ref, b_ref, o_ref):
    del idx_ref
    acc = jnp.dot(w_ref[0], x_ref[0], preferred_element_type=jnp.float32)
    o_ref[0] = acc + b_ref[0]


@jax.jit
def kernel(x, routing_vector, W1, b1, W2, b2, emb, convW, convB, task):
    B = x.shape[0]

    idx2d = pl.pallas_call(
        _routing_body,
        out_shape=jax.ShapeDtypeStruct((B, 1), jnp.int32),
    )(routing_vector, W1, b1.reshape(1, -1), W2, b2.reshape(1, -1), emb)
    idx = idx2d.reshape(B)

    taps = []
    for kh in range(3):
        for kw in range(3):
            taps.append(jax.lax.slice(
                x, (0, 0, kh, kw), (B, _IN_CH, kh + 221, kw + 221), (1, 1, 2, 2)))
    xt = jnp.stack(taps, axis=1).reshape(B, _KDIM, _NPIX)
    xt = jnp.pad(xt, ((0, 0), (0, 0), (0, _NPAD - _NPIX)))

    wmat = jnp.transpose(convW, (0, 1, 3, 4, 2)).reshape(_E, _OUT_CH, _KDIM)
    bias = (convB + jnp.asarray(task, jnp.float32)).reshape(_E, _OUT_CH, 1)

    nt = _NPAD // _TILE_N
    grid_spec = pltpu.PrefetchScalarGridSpec(
        num_scalar_prefetch=1,
        grid=(B, nt),
        in_specs=[
            pl.BlockSpec((1, _KDIM, _TILE_N), lambda bb, t, sidx: (bb, 0, t)),
            pl.BlockSpec((1, _OUT_CH, _KDIM), lambda bb, t, sidx: (sidx[bb], 0, 0)),
            pl.BlockSpec((1, _OUT_CH, 1), lambda bb, t, sidx: (sidx[bb], 0, 0)),
        ],
        out_specs=pl.BlockSpec((1, _OUT_CH, _TILE_N), lambda bb, t, sidx: (bb, 0, t)),
    )
    out = pl.pallas_call(
        _conv_body,
        grid_spec=grid_spec,
        out_shape=jax.ShapeDtypeStruct((B, _OUT_CH, _NPAD), jnp.float32),
    )(idx, xt, wmat, bias)

    return out[:, :, :_NPIX].reshape(B, _OUT_CH, _OH, _OH)


# speedup vs baseline: 17.8756x; 17.8756x over previous
import jax
import jax.numpy as jnp
from jax.experimental import pallas as pl
from jax.experimental.pallas import tpu as pltpu

_E = 8
_CH = 96
_OH = 111
_RO = 8
_RT = 14
_WF = 222
_WO = 128


def _routing_body(rv_ref, w1_ref, b1_ref, w2_ref, b2_ref, emb_ref, idx_ref):
    h = jnp.dot(rv_ref[...], w1_ref[...], preferred_element_type=jnp.float32)
    h = jnp.maximum(h + b1_ref[...], 0.0)
    r = jnp.dot(h, w2_ref[...], preferred_element_type=jnp.float32) + b2_ref[...]
    rn = r / jnp.maximum(jnp.sqrt(jnp.sum(r * r, axis=-1, keepdims=True)), 1e-8)
    e = emb_ref[...]
    en = e / jnp.maximum(jnp.sqrt(jnp.sum(e * e, axis=-1, keepdims=True)), 1e-8)
    sim = jax.lax.dot_general(rn, en, (((1,), (1,)), ((), ())),
                              preferred_element_type=jnp.float32)
    m = jnp.max(sim, axis=-1, keepdims=True)
    iota = jax.lax.broadcasted_iota(jnp.int32, sim.shape, 1)
    cand = jnp.where(sim >= m, iota, jnp.int32(_E))
    idx_ref[...] = jnp.min(cand, axis=-1, keepdims=True)


def _conv_body(idx_ref, xa_ref, xb_ref, w_ref, b_ref, o_ref):
    del idx_ref
    ri = jax.lax.broadcasted_iota(jnp.int32, (_WF, _WO), 0)
    ci = jax.lax.broadcasted_iota(jnp.int32, (_WF, _WO), 1)
    sel = (ri == 2 * ci).astype(jnp.float32)
    bias = b_ref[0]
    for oh in range(_RO):
        acc = None
        for kh in range(3):
            lr = 2 * oh + kh
            row = xa_ref[0, lr] if lr < 2 * _RO else xb_ref[0, 0]
            for kw in range(3):
                part = jnp.dot(w_ref[0, kh * 3 + kw], row[:, kw:kw + _WF],
                               preferred_element_type=jnp.float32)
                acc = part if acc is None else acc + part
        out = jnp.dot(acc, sel, preferred_element_type=jnp.float32)
        o_ref[0, oh] = out + bias


@jax.jit
def kernel(x, routing_vector, W1, b1, W2, b2, emb, convW, convB, task):
    B = x.shape[0]

    idx2d = pl.pallas_call(
        _routing_body,
        out_shape=jax.ShapeDtypeStruct((B, 1), jnp.int32),
    )(routing_vector, W1, b1.reshape(1, -1), W2, b2.reshape(1, -1), emb)
    idx = idx2d.reshape(B)

    xt = jnp.pad(jnp.transpose(x, (0, 2, 1, 3)), ((0, 0), (0, 16), (0, 0), (0, 0)))

    w9 = jnp.transpose(convW, (0, 3, 4, 1, 2)).reshape(_E, 9, _CH, _CH)
    bias = (convB + jnp.asarray(task, jnp.float32)).reshape(_E, _CH, 1)

    grid_spec = pltpu.PrefetchScalarGridSpec(
        num_scalar_prefetch=1,
        grid=(B, _RT),
        in_specs=[
            pl.BlockSpec((1, 2 * _RO, _CH, 224), lambda bb, r, sidx: (bb, r, 0, 0)),
            pl.BlockSpec((1, 2 * _RO, _CH, 224), lambda bb, r, sidx: (bb, r + 1, 0, 0)),
            pl.BlockSpec((1, 9, _CH, _CH), lambda bb, r, sidx: (sidx[bb], 0, 0, 0)),
            pl.BlockSpec((1, _CH, 1), lambda bb, r, sidx: (sidx[bb], 0, 0)),
        ],
        out_specs=pl.BlockSpec((1, _RO, _CH, _WO), lambda bb, r, sidx: (bb, r, 0, 0)),
    )
    out = pl.pallas_call(
        _conv_body,
        grid_spec=grid_spec,
        out_shape=jax.ShapeDtypeStruct((B, _RT * _RO, _CH, _WO), jnp.float32),
    )(idx, xt, xt, w9, bias)

    return jnp.transpose(out, (0, 2, 1, 3))[:, :, :_OH, :_OH]


# speedup vs baseline: 22.5926x; 1.2639x over previous
import jax
import jax.numpy as jnp
from jax.experimental import pallas as pl
from jax.experimental.pallas import tpu as pltpu

_E = 8
_CH = 96
_OH = 111
_RO = 8
_RT = 14
_W = 224
_WO = 128
_M3 = 3 * _CH
_MR = _RO * _CH


def _routing_body(rv_ref, w1_ref, b1_ref, w2_ref, b2_ref, emb_ref, idx_ref):
    h = jnp.dot(rv_ref[...], w1_ref[...], preferred_element_type=jnp.float32)
    h = jnp.maximum(h + b1_ref[...], 0.0)
    r = jnp.dot(h, w2_ref[...], preferred_element_type=jnp.float32) + b2_ref[...]
    rn = r / jnp.maximum(jnp.sqrt(jnp.sum(r * r, axis=-1, keepdims=True)), 1e-8)
    e = emb_ref[...]
    en = e / jnp.maximum(jnp.sqrt(jnp.sum(e * e, axis=-1, keepdims=True)), 1e-8)
    sim = jax.lax.dot_general(rn, en, (((1,), (1,)), ((), ())),
                              preferred_element_type=jnp.float32)
    m = jnp.max(sim, axis=-1, keepdims=True)
    iota = jax.lax.broadcasted_iota(jnp.int32, sim.shape, 1)
    cand = jnp.where(sim >= m, iota, jnp.int32(_E))
    idx_ref[...] = jnp.min(cand, axis=-1, keepdims=True)


def _conv_body(idx_ref, xa_ref, xb_ref, w_ref, b_ref, o_ref, s_ref):
    del idx_ref
    wall = w_ref[0]
    for oh in range(_RO):
        lr = 2 * oh
        if lr + 3 <= 2 * _RO:
            r3 = xa_ref[0, lr:lr + 3].reshape(_M3, _W)
        else:
            r3 = jnp.concatenate(
                [xa_ref[0, lr:lr + 2].reshape(2 * _CH, _W),
                 xb_ref[0, 0:1].reshape(_CH, _W)], axis=0)
        t = jnp.dot(wall, r3, preferred_element_type=jnp.float32)
        for kw in range(3):
            s_ref[kw, oh * _CH:(oh + 1) * _CH, :] = t[kw * _CH:(kw + 1) * _CH, :]
    acc = None
    for kw in range(3):
        ri = jax.lax.broadcasted_iota(jnp.int32, (_W, _WO), 0)
        ci = jax.lax.broadcasted_iota(jnp.int32, (_W, _WO), 1)
        sel = (ri == 2 * ci + kw).astype(jnp.float32)
        d = jnp.dot(s_ref[kw], sel, preferred_element_type=jnp.float32)
        acc = d if acc is None else acc + d
    o_ref[0] = acc.reshape(_RO, _CH, _WO) + b_ref[...]


@jax.jit
def kernel(x, routing_vector, W1, b1, W2, b2, emb, convW, convB, task):
    B = x.shape[0]

    idx2d = pl.pallas_call(
        _routing_body,
        out_shape=jax.ShapeDtypeStruct((B, 1), jnp.int32),
    )(routing_vector, W1, b1.reshape(1, -1), W2, b2.reshape(1, -1), emb)
    idx = idx2d.reshape(B)

    xt = jnp.pad(jnp.transpose(x, (0, 2, 1, 3)), ((0, 0), (0, 16), (0, 0), (0, 0)))

    w3 = jnp.transpose(convW, (0, 4, 1, 3, 2)).reshape(_E, _M3, _M3)
    bias = (convB + jnp.asarray(task, jnp.float32)).reshape(_E, _CH, 1)

    grid_spec = pltpu.PrefetchScalarGridSpec(
        num_scalar_prefetch=1,
        grid=(B, _RT),
        in_specs=[
            pl.BlockSpec((1, 2 * _RO, _CH, _W), lambda bb, r, sidx: (bb, r, 0, 0)),
            pl.BlockSpec((1, 2 * _RO, _CH, _W), lambda bb, r, sidx: (bb, r + 1, 0, 0)),
            pl.BlockSpec((1, _M3, _M3), lambda bb, r, sidx: (sidx[bb], 0, 0)),
            pl.BlockSpec((1, _CH, 1), lambda bb, r, sidx: (sidx[bb], 0, 0)),
        ],
        out_specs=pl.BlockSpec((1, _RO, _CH, _WO), lambda bb, r, sidx: (bb, r, 0, 0)),
        scratch_shapes=[pltpu.VMEM((3, _MR, _W), jnp.float32)],
    )
    out = pl.pallas_call(
        _conv_body,
        grid_spec=grid_spec,
        out_shape=jax.ShapeDtypeStruct((B, _RT * _RO, _CH, _WO), jnp.float32),
    )(idx, xt, xt, w3, bias)

    return jnp.transpose(out, (0, 2, 1, 3))[:, :, :_OH, :_OH]


# speedup vs baseline: 28.2583x; 1.2508x over previous
import jax
import jax.numpy as jnp
from jax.experimental import pallas as pl
from jax.experimental.pallas import tpu as pltpu

_E = 8
_CH = 96
_OH = 111
_RO = 8
_RT = 14
_W = 224
_WO = 128
_M3 = 3 * _CH
_MR = _RO * _CH


def _routing_body(rv_ref, w1_ref, b1_ref, w2_ref, b2_ref, emb_ref, idx_ref):
    h = jnp.dot(rv_ref[...], w1_ref[...], preferred_element_type=jnp.float32)
    h = jnp.maximum(h + b1_ref[...], 0.0)
    r = jnp.dot(h, w2_ref[...], preferred_element_type=jnp.float32) + b2_ref[...]
    rn = r / jnp.maximum(jnp.sqrt(jnp.sum(r * r, axis=-1, keepdims=True)), 1e-8)
    e = emb_ref[...]
    en = e / jnp.maximum(jnp.sqrt(jnp.sum(e * e, axis=-1, keepdims=True)), 1e-8)
    sim = jax.lax.dot_general(rn, en, (((1,), (1,)), ((), ())),
                              preferred_element_type=jnp.float32)
    m = jnp.max(sim, axis=-1, keepdims=True)
    iota = jax.lax.broadcasted_iota(jnp.int32, sim.shape, 1)
    cand = jnp.where(sim >= m, iota, jnp.int32(_E))
    idx_ref[...] = jnp.min(cand, axis=-1, keepdims=True)


def _conv_body(idx_ref, xa_ref, xb_ref, w_ref, b_ref, o_ref, s_ref):
    del idx_ref
    wall = w_ref[0]
    for oh in range(_RO):
        lr = 2 * oh
        if lr + 3 <= 2 * _RO:
            r3 = xa_ref[0, lr:lr + 3].reshape(_M3, _W)
        else:
            r3 = jnp.concatenate(
                [xa_ref[0, lr:lr + 2].reshape(2 * _CH, _W),
                 xb_ref[0, 0:1].reshape(_CH, _W)], axis=0)
        t = jnp.dot(wall, r3, preferred_element_type=jnp.float32)
        for kw in range(3):
            s_ref[kw, oh * _CH:(oh + 1) * _CH, :] = t[kw * _CH:(kw + 1) * _CH, :]
    acc = None
    for kw in range(3):
        ri = jax.lax.broadcasted_iota(jnp.int32, (_W, _WO), 0)
        ci = jax.lax.broadcasted_iota(jnp.int32, (_W, _WO), 1)
        sel = (ri == 2 * ci + kw).astype(jnp.float32)
        d = jnp.dot(s_ref[kw], sel, preferred_element_type=jnp.float32)
        acc = d if acc is None else acc + d
    o_ref[0] = acc.reshape(_RO, _CH, _WO) + b_ref[...]


@jax.jit
def kernel(x, routing_vector, W1, b1, W2, b2, emb, convW, convB, task):
    B = x.shape[0]

    idx2d = pl.pallas_call(
        _routing_body,
        out_shape=jax.ShapeDtypeStruct((B, 1), jnp.int32),
    )(routing_vector, W1, b1.reshape(1, -1), W2, b2.reshape(1, -1), emb)
    idx = idx2d.reshape(B)

    xt = jnp.transpose(x, (0, 2, 1, 3)).astype(jnp.bfloat16)

    w3 = jnp.transpose(convW, (0, 4, 1, 3, 2)).reshape(_E, _M3, _M3).astype(jnp.bfloat16)
    bias = (convB + jnp.asarray(task, jnp.float32)).reshape(_E, _CH, 1)

    grid_spec = pltpu.PrefetchScalarGridSpec(
        num_scalar_prefetch=1,
        grid=(B, _RT),
        in_specs=[
            pl.BlockSpec((1, 2 * _RO, _CH, _W), lambda bb, r, sidx: (bb, r, 0, 0)),
            pl.BlockSpec((1, 2 * _RO, _CH, _W),
                         lambda bb, r, sidx: (bb, jnp.minimum(r + 1, _RT - 1), 0, 0)),
            pl.BlockSpec((1, _M3, _M3), lambda bb, r, sidx: (sidx[bb], 0, 0)),
            pl.BlockSpec((1, _CH, 1), lambda bb, r, sidx: (sidx[bb], 0, 0)),
        ],
        out_specs=pl.BlockSpec((1, _RO, _CH, _WO), lambda bb, r, sidx: (bb, r, 0, 0)),
        scratch_shapes=[pltpu.VMEM((3, _MR, _W), jnp.float32)],
    )
    out = pl.pallas_call(
        _conv_body,
        grid_spec=grid_spec,
        out_shape=jax.ShapeDtypeStruct((B, _RT * _RO, _CH, _WO), jnp.float32),
    )(idx, xt, xt, w3, bias)

    return jnp.transpose(out, (0, 2, 1, 3))[:, :, :_OH, :_OH]


# speedup vs baseline: 30.8752x; 1.0926x over previous
import jax
import jax.numpy as jnp
from jax.experimental import pallas as pl
from jax.experimental.pallas import tpu as pltpu

_E = 8
_CH = 96
_OH = 111
_RO = 16
_RT = 7
_W = 224
_WO = 128
_M3 = 3 * _CH
_MR = _RO * _CH


def _conv_body(rv_ref, w1_ref, b1_ref, w2_ref, b2_ref, emb_ref,
               xa_ref, xb_ref, wall_ref, ball_ref,
               o_ref, idx_ref, wsel_ref, bsel_ref, s_ref):
    b = pl.program_id(0)
    r = pl.program_id(1)

    @pl.when(jnp.logical_and(b == 0, r == 0))
    def _routing():
        h = jnp.dot(rv_ref[...], w1_ref[...], preferred_element_type=jnp.float32)
        h = jnp.maximum(h + b1_ref[...], 0.0)
        rr = jnp.dot(h, w2_ref[...], preferred_element_type=jnp.float32) + b2_ref[...]
        rn = rr / jnp.maximum(jnp.sqrt(jnp.sum(rr * rr, axis=-1, keepdims=True)), 1e-8)
        e = emb_ref[...]
        en = e / jnp.maximum(jnp.sqrt(jnp.sum(e * e, axis=-1, keepdims=True)), 1e-8)
        sim = jax.lax.dot_general(rn, en, (((1,), (1,)), ((), ())),
                                  preferred_element_type=jnp.float32)
        m = jnp.max(sim, axis=-1, keepdims=True)
        iota = jax.lax.broadcasted_iota(jnp.int32, sim.shape, 1)
        cand = jnp.where(sim >= m, iota, jnp.int32(_E))
        idx_ref[...] = jnp.min(cand, axis=-1, keepdims=True)

    @pl.when(r == 0)
    def _gather_expert():
        myidx = idx_ref[pl.ds(b, 1), :]
        wacc = jnp.zeros((_M3, _M3), jnp.bfloat16)
        bacc = jnp.zeros((_CH, 1), jnp.float32)
        for e in range(_E):
            hit = (myidx == e).reshape(1, 1)
            wacc = wacc + jnp.where(hit, wall_ref[e], jnp.bfloat16(0))
            bacc = bacc + jnp.where(hit, ball_ref[e], 0.0)
        wsel_ref[...] = wacc
        bsel_ref[...] = bacc

    wall = wsel_ref[...]
    for oh in range(_RO):
        lr = 2 * oh
        if lr + 3 <= 2 * _RO:
            r3 = xa_ref[0, lr:lr + 3].reshape(_M3, _W)
        else:
            r3 = jnp.concatenate(
                [xa_ref[0, lr:lr + 2].reshape(2 * _CH, _W),
                 xb_ref[0, 0:1].reshape(_CH, _W)], axis=0)
        t = jnp.dot(wall, r3, preferred_element_type=jnp.float32)
        tb = t.astype(jnp.bfloat16)
        for kw in range(3):
            s_ref[kw, oh * _CH:(oh + 1) * _CH, :] = tb[kw * _CH:(kw + 1) * _CH, :]
    acc = None
    for kw in range(3):
        ri = jax.lax.broadcasted_iota(jnp.int32, (_W, _WO), 0)
        ci = jax.lax.broadcasted_iota(jnp.int32, (_W, _WO), 1)
        sel = (ri == 2 * ci + kw).astype(jnp.bfloat16)
        d = jnp.dot(s_ref[kw], sel, preferred_element_type=jnp.float32)
        acc = d if acc is None else acc + d
    o_ref[0] = acc.reshape(_RO, _CH, _WO) + bsel_ref[...]


@jax.jit
def kernel(x, routing_vector, W1, b1, W2, b2, emb, convW, convB, task):
    B = x.shape[0]

    xt = jnp.transpose(x, (0, 2, 1, 3)).astype(jnp.bfloat16)

    w3 = jnp.transpose(convW, (0, 4, 1, 3, 2)).reshape(_E, _M3, _M3).astype(jnp.bfloat16)
    bias = (convB + jnp.asarray(task, jnp.float32)).reshape(_E, _CH, 1)

    full = lambda bb, r: (0, 0)
    out = pl.pallas_call(
        _conv_body,
        grid=(B, _RT),
        in_specs=[
            pl.BlockSpec(routing_vector.shape, full),
            pl.BlockSpec(W1.shape, full),
            pl.BlockSpec((1, 128), full),
            pl.BlockSpec(W2.shape, full),
            pl.BlockSpec((1, 64), full),
            pl.BlockSpec(emb.shape, full),
            pl.BlockSpec((1, 2 * _RO, _CH, _W), lambda bb, r: (bb, r, 0, 0)),
            pl.BlockSpec((1, 2 * _RO, _CH, _W),
                         lambda bb, r: (bb, jnp.minimum(r + 1, _RT - 1), 0, 0)),
            pl.BlockSpec((_E, _M3, _M3), lambda bb, r: (0, 0, 0)),
            pl.BlockSpec((_E, _CH, 1), lambda bb, r: (0, 0, 0)),
        ],
        out_specs=pl.BlockSpec((1, _RO, _CH, _WO), lambda bb, r: (bb, r, 0, 0)),
        scratch_shapes=[
            pltpu.VMEM((B, 1), jnp.int32),
            pltpu.VMEM((_M3, _M3), jnp.bfloat16),
            pltpu.VMEM((_CH, 1), jnp.float32),
            pltpu.VMEM((3, _MR, _W), jnp.bfloat16),
        ],
        out_shape=jax.ShapeDtypeStruct((B, _RT * _RO, _CH, _WO), jnp.float32),
    )(routing_vector, W1, b1.reshape(1, -1), W2, b2.reshape(1, -1), emb,
      xt, xt, w3, bias)

    return jnp.transpose(out, (0, 2, 1, 3))[:, :, :_OH, :_OH]
